# Initial kernel scaffold; baseline (speedup 1.0000x reference)
import functools

import jax
import jax.numpy as jnp
from jax import lax
from jax.experimental import pallas as pl
from jax.experimental.pallas import tpu as pltpu
from jax.experimental.pallas import tpu_sc as plsc

N_NODES = 100000
N_EDGES = 3200000
F = 16
NC, NS = 2, 16
NW = NC * NS
ROW = 128
SUPK = 16
RW = 784
NSUP = RW // SUPK
PE = NW * RW * ROW
STRIPE = 6272
NPAD = NS * STRIPE
DUMP = N_NODES
BLK = 2000
GRID = N_NODES // BLK


def _sc_mesh():
    return plsc.VectorSubcoreMesh(core_axis_name="c", subcore_axis_name="s")


def _sc_degree(dst2d):
    def body(dst_hbm, out_hbm, didx, buf, deg_sh):
        cid = lax.axis_index("c")
        tid = lax.axis_index("s")
        wid = cid * NS + tid

        def fill(val):
            def w(i, c):
                buf[pl.ds(i * F, F)] = jnp.full((F,), val, jnp.float32)
                return c
            lax.fori_loop(0, ROW // F, w, 0)

        fill(0.0)

        def zcopy(t, c):
            pltpu.sync_copy(buf, deg_sh.at[pl.ds(tid * STRIPE + t * ROW, ROW)])
            return c

        lax.fori_loop(0, STRIPE // ROW, zcopy, 0)
        fill(1.0)
        plsc.subcore_barrier()

        base = wid * RW

        def sup(s, c):
            pltpu.sync_copy(dst_hbm.at[pl.ds(base + s * SUPK, SUPK)], didx)
            for j in range(SUPK):
                pltpu.sync_copy(buf, deg_sh.at[didx.at[j]], add=True)
            return c

        lax.fori_loop(0, NSUP, sup, 0)
        plsc.subcore_barrier()
        pltpu.sync_copy(deg_sh.at[pl.ds(tid * STRIPE, STRIPE)],
                        out_hbm.at[cid, pl.ds(tid * STRIPE, STRIPE)])

    return pl.kernel(
        body,
        out_type=jax.ShapeDtypeStruct((NC, NPAD), jnp.float32),
        mesh=_sc_mesh(),
        scratch_types=[
            pltpu.VMEM((SUPK, ROW), jnp.int32),
            pltpu.VMEM((ROW,), jnp.float32),
            pltpu.VMEM_SHARED((NPAD,), jnp.float32),
        ],
    )(dst2d)


def _sc_scatter(src2d, dst2d, g):
    def body(src_hbm, dst_hbm, g_hbm, out_hbm, sidx, didx, rowbuf, acc_sh, gsem):
        cid = lax.axis_index("c")
        tid = lax.axis_index("s")
        wid = cid * NS + tid

        def zrow(i, c):
            rowbuf[i, :] = jnp.zeros((F,), jnp.float32)
            return c

        lax.fori_loop(0, ROW, zrow, 0)

        def zcopy(t, c):
            pltpu.sync_copy(rowbuf, acc_sh.at[pl.ds(tid * STRIPE + t * ROW, ROW)])
            return c

        lax.fori_loop(0, STRIPE // ROW, zcopy, 0)
        plsc.subcore_barrier()

        base = wid * RW

        def sup(s, c):
            r0 = base + s * SUPK
            pltpu.sync_copy(src_hbm.at[pl.ds(r0, SUPK)], sidx)
            pltpu.sync_copy(dst_hbm.at[pl.ds(r0, SUPK)], didx)
            for j in range(SUPK):
                pltpu.async_copy(g_hbm.at[sidx.at[j]], rowbuf, gsem).wait()
                pltpu.sync_copy(rowbuf, acc_sh.at[didx.at[j]], add=True)
            return c

        lax.fori_loop(0, NSUP, sup, 0)
        plsc.subcore_barrier()
        pltpu.sync_copy(acc_sh.at[pl.ds(tid * STRIPE, STRIPE)],
                        out_hbm.at[cid, pl.ds(tid * STRIPE, STRIPE)])

    return pl.kernel(
        body,
        out_type=jax.ShapeDtypeStruct((NC, NPAD, F), jnp.float32),
        mesh=_sc_mesh(),
        scratch_types=[
            pltpu.VMEM((SUPK, ROW), jnp.int32),
            pltpu.VMEM((SUPK, ROW), jnp.int32),
            pltpu.VMEM((ROW, F), jnp.float32),
            pltpu.VMEM_SHARED((NPAD, F), jnp.float32),
            pltpu.SemaphoreType.DMA,
        ],
    )(src2d, dst2d, g)


def _dinv_block(degp):
    return lax.rsqrt(degp[0, :] + degp[1, :] + 1.0)


def _tc_embed(x, S, R, We1a, We1b, be1, We2, be2, Wg1, degp):
    def body(x_ref, s_ref, r_ref, we1a, we1b, be1_r, we2, be2_r, wg1, degp_ref,
             g_ref):
        q = r_ref[...] @ we1b[...]
        h = jnp.tanh(x_ref[...] @ we1a[...] + s_ref[...] @ q + be1_r[...])
        h = jnp.tanh(h @ we2[...] + be2_r[...])
        dinv = _dinv_block(degp_ref[...])
        g_ref[...] = (h @ wg1[...]) * dinv[:, None]

    return pl.pallas_call(
        body,
        grid=(GRID,),
        in_specs=[
            pl.BlockSpec((BLK, 42), lambda i: (i, 0)),
            pl.BlockSpec((BLK, 128), lambda i: (i, 0)),
            pl.BlockSpec((128, 3), lambda i: (0, 0)),
            pl.BlockSpec((42, 32), lambda i: (0, 0)),
            pl.BlockSpec((3, 32), lambda i: (0, 0)),
            pl.BlockSpec((1, 32), lambda i: (0, 0)),
            pl.BlockSpec((32, F), lambda i: (0, 0)),
            pl.BlockSpec((1, F), lambda i: (0, 0)),
            pl.BlockSpec((F, F), lambda i: (0, 0)),
            pl.BlockSpec((NC, BLK), lambda i: (0, i)),
        ],
        out_specs=pl.BlockSpec((BLK, F), lambda i: (i, 0)),
        out_shape=jax.ShapeDtypeStruct((N_NODES, F), jnp.float32),
    )(x, S, R, We1a, We1b, be1, We2, be2, Wg1, degp)


def _tc_mid(accp, g1, degp, bg1, Wg2):
    def body(accp_ref, g1_ref, degp_ref, bg1_r, wg2, g2_ref):
        dinv = _dinv_block(degp_ref[...])
        acc = accp_ref[0, :, :] + accp_ref[1, :, :] + g1_ref[...]
        h = jax.nn.relu(acc * dinv[:, None] + bg1_r[...])
        g2_ref[...] = (h @ wg2[...]) * dinv[:, None]

    return pl.pallas_call(
        body,
        grid=(GRID,),
        in_specs=[
            pl.BlockSpec((NC, BLK, F), lambda i: (0, i, 0)),
            pl.BlockSpec((BLK, F), lambda i: (i, 0)),
            pl.BlockSpec((NC, BLK), lambda i: (0, i)),
            pl.BlockSpec((1, F), lambda i: (0, 0)),
            pl.BlockSpec((F, F), lambda i: (0, 0)),
        ],
        out_specs=pl.BlockSpec((BLK, F), lambda i: (i, 0)),
        out_shape=jax.ShapeDtypeStruct((N_NODES, F), jnp.float32),
    )(accp, g1, degp, bg1, Wg2)


def _tc_pred(accp, g2, degp, bg2, Wp1, bp1, Wp2, bp2, priors):
    def body(accp_ref, g2_ref, degp_ref, bg2_r, wp1, bp1_r, wp2, bp2_r, pri_ref,
             out_ref):
        dinv = _dinv_block(degp_ref[...])
        acc = accp_ref[0, :, :] + accp_ref[1, :, :] + g2_ref[...]
        h = jax.nn.relu(acc * dinv[:, None] + bg2_r[...])
        o = jnp.tanh(h @ wp1[...] + bp1_r[...])
        o = jnp.tanh(o @ wp2[...] + bp2_r[...])
        out_ref[...] = o + pri_ref[...]

    return pl.pallas_call(
        body,
        grid=(GRID,),
        in_specs=[
            pl.BlockSpec((NC, BLK, F), lambda i: (0, i, 0)),
            pl.BlockSpec((BLK, F), lambda i: (i, 0)),
            pl.BlockSpec((NC, BLK), lambda i: (0, i)),
            pl.BlockSpec((1, F), lambda i: (0, 0)),
            pl.BlockSpec((F, 8), lambda i: (0, 0)),
            pl.BlockSpec((1, 8), lambda i: (0, 0)),
            pl.BlockSpec((8, 1), lambda i: (0, 0)),
            pl.BlockSpec((1, 1), lambda i: (0, 0)),
            pl.BlockSpec((BLK, 1), lambda i: (i, 0)),
        ],
        out_specs=pl.BlockSpec((BLK, 1), lambda i: (i, 0)),
        out_shape=jax.ShapeDtypeStruct((N_NODES, 1), jnp.float32),
    )(accp, g2, degp, bg2, Wp1, bp1, Wp2, bp2, priors)


def kernel(x, edge_index, priors, S, R, We1, be1, We2, be2, Wg1, bg1, Wg2, bg2,
           Wp1, bp1, Wp2, bp2):
    src = edge_index[0].astype(jnp.int32)
    dst = edge_index[1].astype(jnp.int32)
    pad = PE - N_EDGES
    src2d = jnp.concatenate([src, jnp.zeros((pad,), jnp.int32)]).reshape(-1, ROW)
    dst2d = jnp.concatenate([dst, jnp.full((pad,), DUMP, jnp.int32)]).reshape(-1, ROW)

    degp = _sc_degree(dst2d)
    g1 = _tc_embed(x, S, R, We1[:42], We1[42:], be1.reshape(1, -1),
                   We2, be2.reshape(1, -1), Wg1, degp)
    acc1 = _sc_scatter(src2d, dst2d, g1)
    g2 = _tc_mid(acc1, g1, degp, bg1.reshape(1, -1), Wg2)
    acc2 = _sc_scatter(src2d, dst2d, g2)
    return _tc_pred(acc2, g2, degp, bg2.reshape(1, -1), Wp1, bp1.reshape(1, -1),
                    Wp2, bp2.reshape(1, -1), priors)


# speedup vs baseline: 35.0900x; 35.0900x over previous
import functools

import jax
import jax.numpy as jnp
from jax import lax
from jax.experimental import pallas as pl
from jax.experimental.pallas import tpu as pltpu
from jax.experimental.pallas import tpu_sc as plsc

N_NODES = 100000
N_EDGES = 3200000
F = 16
NC, NS = 2, 16
NW = NC * NS
ROW = 128
SUPK = 16
RW = 784
NSUP = RW // SUPK
PE = NW * RW * ROW
STRIPE = 6272
NPAD = NS * STRIPE
DUMP = N_NODES
BLK = 2000
GRID = N_NODES // BLK


def _sc_mesh():
    return plsc.VectorSubcoreMesh(core_axis_name="c", subcore_axis_name="s")


def _sc_degree(dst2d):
    def body(dst_hbm, out_hbm, didx, buf, deg_sh):
        cid = lax.axis_index("c")
        tid = lax.axis_index("s")
        wid = cid * NS + tid

        def fill(val):
            def w(i, c):
                buf[pl.ds(i * F, F)] = jnp.full((F,), val, jnp.float32)
                return c
            lax.fori_loop(0, ROW // F, w, 0)

        fill(0.0)

        def zcopy(t, c):
            pltpu.sync_copy(buf, deg_sh.at[pl.ds(tid * STRIPE + t * ROW, ROW)])
            return c

        lax.fori_loop(0, STRIPE // ROW, zcopy, 0)
        fill(1.0)
        plsc.subcore_barrier()

        base = wid * RW

        def sup(s, c):
            pltpu.sync_copy(dst_hbm.at[pl.ds(base + s * SUPK, SUPK)], didx)
            for j in range(SUPK):
                pltpu.sync_copy(buf, deg_sh.at[didx.at[j]], add=True)
            return c

        lax.fori_loop(0, NSUP, sup, 0)
        plsc.subcore_barrier()
        pltpu.sync_copy(deg_sh.at[pl.ds(tid * STRIPE, STRIPE)],
                        out_hbm.at[cid, pl.ds(tid * STRIPE, STRIPE)])

    return pl.kernel(
        body,
        out_type=jax.ShapeDtypeStruct((NC, NPAD), jnp.float32),
        mesh=_sc_mesh(),
        compiler_params=pltpu.CompilerParams(use_tc_tiling_on_sc=False),
        scratch_types=[
            pltpu.VMEM((SUPK, ROW), jnp.int32),
            pltpu.VMEM((ROW,), jnp.float32),
            pltpu.VMEM_SHARED((NPAD,), jnp.float32),
        ],
    )(dst2d)


def _sc_scatter(src2d, dst2d, g):
    def body(src_hbm, dst_hbm, g_hbm, out_hbm, sidx, didx, rowbuf, acc_sh, gsem):
        cid = lax.axis_index("c")
        tid = lax.axis_index("s")
        wid = cid * NS + tid

        def zrow(i, c):
            rowbuf[i, :] = jnp.zeros((F,), jnp.float32)
            return c

        lax.fori_loop(0, ROW, zrow, 0)

        def zcopy(t, c):
            pltpu.sync_copy(rowbuf, acc_sh.at[pl.ds(tid * STRIPE + t * ROW, ROW)])
            return c

        lax.fori_loop(0, STRIPE // ROW, zcopy, 0)
        plsc.subcore_barrier()

        base = wid * RW

        def sup(s, c):
            r0 = base + s * SUPK
            pltpu.sync_copy(src_hbm.at[pl.ds(r0, SUPK)], sidx)
            pltpu.sync_copy(dst_hbm.at[pl.ds(r0, SUPK)], didx)
            for j in range(SUPK):
                pltpu.async_copy(g_hbm.at[sidx.at[j]], rowbuf, gsem).wait()
                pltpu.sync_copy(rowbuf, acc_sh.at[didx.at[j]], add=True)
            return c

        lax.fori_loop(0, NSUP, sup, 0)
        plsc.subcore_barrier()
        pltpu.sync_copy(acc_sh.at[pl.ds(tid * STRIPE, STRIPE)],
                        out_hbm.at[cid, pl.ds(tid * STRIPE, STRIPE)])

    return pl.kernel(
        body,
        out_type=jax.ShapeDtypeStruct((NC, NPAD, F), jnp.float32),
        mesh=_sc_mesh(),
        compiler_params=pltpu.CompilerParams(use_tc_tiling_on_sc=False),
        scratch_types=[
            pltpu.VMEM((SUPK, ROW), jnp.int32),
            pltpu.VMEM((SUPK, ROW), jnp.int32),
            pltpu.VMEM((ROW, F), jnp.float32),
            pltpu.VMEM_SHARED((NPAD, F), jnp.float32),
            pltpu.SemaphoreType.DMA,
        ],
    )(src2d, dst2d, g)


def _dinv_block(degp):
    return lax.rsqrt(degp[0, :, 0] + degp[1, :, 0] + 1.0)


def _tc_embed(x, S, R, We1a, We1b, be1, We2, be2, Wg1, degp):
    def body(x_ref, s_ref, r_ref, we1a, we1b, be1_r, we2, be2_r, wg1, degp_ref,
             g_ref):
        q = r_ref[...] @ we1b[...]
        h = jnp.tanh(x_ref[...] @ we1a[...] + s_ref[...] @ q + be1_r[...])
        h = jnp.tanh(h @ we2[...] + be2_r[...])
        dinv = _dinv_block(degp_ref[...])
        g_ref[...] = (h @ wg1[...]) * dinv[:, None]

    return pl.pallas_call(
        body,
        grid=(GRID,),
        in_specs=[
            pl.BlockSpec((BLK, 42), lambda i: (i, 0)),
            pl.BlockSpec((BLK, 128), lambda i: (i, 0)),
            pl.BlockSpec((128, 3), lambda i: (0, 0)),
            pl.BlockSpec((42, 32), lambda i: (0, 0)),
            pl.BlockSpec((3, 32), lambda i: (0, 0)),
            pl.BlockSpec((1, 32), lambda i: (0, 0)),
            pl.BlockSpec((32, F), lambda i: (0, 0)),
            pl.BlockSpec((1, F), lambda i: (0, 0)),
            pl.BlockSpec((F, F), lambda i: (0, 0)),
            pl.BlockSpec((NC, BLK, 1), lambda i: (0, i, 0)),
        ],
        out_specs=pl.BlockSpec((BLK, F), lambda i: (i, 0)),
        out_shape=jax.ShapeDtypeStruct((N_NODES, F), jnp.float32),
    )(x, S, R, We1a, We1b, be1, We2, be2, Wg1, degp)


def _tc_mid(accp, g1, degp, bg1, Wg2):
    def body(accp_ref, g1_ref, degp_ref, bg1_r, wg2, g2_ref):
        dinv = _dinv_block(degp_ref[...])
        acc = accp_ref[0, :, :] + accp_ref[1, :, :] + g1_ref[...]
        h = jax.nn.relu(acc * dinv[:, None] + bg1_r[...])
        g2_ref[...] = (h @ wg2[...]) * dinv[:, None]

    return pl.pallas_call(
        body,
        grid=(GRID,),
        in_specs=[
            pl.BlockSpec((NC, BLK, F), lambda i: (0, i, 0)),
            pl.BlockSpec((BLK, F), lambda i: (i, 0)),
            pl.BlockSpec((NC, BLK, 1), lambda i: (0, i, 0)),
            pl.BlockSpec((1, F), lambda i: (0, 0)),
            pl.BlockSpec((F, F), lambda i: (0, 0)),
        ],
        out_specs=pl.BlockSpec((BLK, F), lambda i: (i, 0)),
        out_shape=jax.ShapeDtypeStruct((N_NODES, F), jnp.float32),
    )(accp, g1, degp, bg1, Wg2)


def _tc_pred(accp, g2, degp, bg2, Wp1, bp1, Wp2, bp2, priors):
    def body(accp_ref, g2_ref, degp_ref, bg2_r, wp1, bp1_r, wp2, bp2_r, pri_ref,
             out_ref):
        dinv = _dinv_block(degp_ref[...])
        acc = accp_ref[0, :, :] + accp_ref[1, :, :] + g2_ref[...]
        h = jax.nn.relu(acc * dinv[:, None] + bg2_r[...])
        o = jnp.tanh(h @ wp1[...] + bp1_r[...])
        o = jnp.tanh(o @ wp2[...] + bp2_r[...])
        out_ref[...] = o + pri_ref[...]

    return pl.pallas_call(
        body,
        grid=(GRID,),
        in_specs=[
            pl.BlockSpec((NC, BLK, F), lambda i: (0, i, 0)),
            pl.BlockSpec((BLK, F), lambda i: (i, 0)),
            pl.BlockSpec((NC, BLK, 1), lambda i: (0, i, 0)),
            pl.BlockSpec((1, F), lambda i: (0, 0)),
            pl.BlockSpec((F, 8), lambda i: (0, 0)),
            pl.BlockSpec((1, 8), lambda i: (0, 0)),
            pl.BlockSpec((8, 1), lambda i: (0, 0)),
            pl.BlockSpec((1, 1), lambda i: (0, 0)),
            pl.BlockSpec((BLK, 1), lambda i: (i, 0)),
        ],
        out_specs=pl.BlockSpec((BLK, 1), lambda i: (i, 0)),
        out_shape=jax.ShapeDtypeStruct((N_NODES, 1), jnp.float32),
    )(accp, g2, degp, bg2, Wp1, bp1, Wp2, bp2, priors)


def kernel(x, edge_index, priors, S, R, We1, be1, We2, be2, Wg1, bg1, Wg2, bg2,
           Wp1, bp1, Wp2, bp2):
    src = edge_index[0].astype(jnp.int32)
    dst = edge_index[1].astype(jnp.int32)
    pad = PE - N_EDGES
    src2d = jnp.concatenate([src, jnp.zeros((pad,), jnp.int32)]).reshape(-1, ROW)
    dst2d = jnp.concatenate([dst, jnp.full((pad,), DUMP, jnp.int32)]).reshape(-1, ROW)

    degp = _sc_degree(dst2d).reshape(NC, NPAD, 1)
    g1 = _tc_embed(x, S, R, We1[:42], We1[42:], be1.reshape(1, -1),
                   We2, be2.reshape(1, -1), Wg1, degp)
    acc1 = _sc_scatter(src2d, dst2d, g1)
    g2 = _tc_mid(acc1, g1, degp, bg1.reshape(1, -1), Wg2)
    acc2 = _sc_scatter(src2d, dst2d, g2)
    return _tc_pred(acc2, g2, degp, bg2.reshape(1, -1), Wp1, bp1.reshape(1, -1),
                    Wp2, bp2.reshape(1, -1), priors)


# speedup vs baseline: 56.4582x; 1.6090x over previous
import functools

import jax
import jax.numpy as jnp
from jax import lax
from jax.experimental import pallas as pl
from jax.experimental.pallas import tpu as pltpu
from jax.experimental.pallas import tpu_sc as plsc

N_NODES = 100000
N_EDGES = 3200000
F = 16
NC, NS = 2, 16
NW = NC * NS
ROW = 128
SUPK = 6
RW = 786
NSUP = RW // SUPK
PE = NW * RW * ROW
STRIPE = 6272
NPAD = NS * STRIPE
DUMP = N_NODES
BLK = 2000
GRID = N_NODES // BLK


def _sc_mesh():
    return plsc.VectorSubcoreMesh(core_axis_name="c", subcore_axis_name="s")


def _sc_degree(dst2d):
    def body(dst_hbm, out_hbm, didx, buf, deg_sh):
        cid = lax.axis_index("c")
        tid = lax.axis_index("s")
        wid = cid * NS + tid

        def fill(val):
            def w(i, c):
                buf[pl.ds(i * F, F)] = jnp.full((F,), val, jnp.float32)
                return c
            lax.fori_loop(0, ROW // F, w, 0)

        fill(0.0)

        def zcopy(t, c):
            pltpu.sync_copy(buf, deg_sh.at[pl.ds(tid * STRIPE + t * ROW, ROW)])
            return c

        lax.fori_loop(0, STRIPE // ROW, zcopy, 0)
        fill(1.0)
        plsc.subcore_barrier()

        base = wid * RW

        def sup(s, c):
            pltpu.sync_copy(dst_hbm.at[pl.ds(base + s * SUPK, SUPK)], didx)
            for j in range(SUPK):
                pltpu.sync_copy(buf, deg_sh.at[didx.at[j]], add=True)
            return c

        lax.fori_loop(0, NSUP, sup, 0)
        plsc.subcore_barrier()
        pltpu.sync_copy(deg_sh.at[pl.ds(tid * STRIPE, STRIPE)],
                        out_hbm.at[cid, pl.ds(tid * STRIPE, STRIPE)])

    return pl.kernel(
        body,
        out_type=jax.ShapeDtypeStruct((NC, NPAD), jnp.float32),
        mesh=_sc_mesh(),
        compiler_params=pltpu.CompilerParams(use_tc_tiling_on_sc=False),
        scratch_types=[
            pltpu.VMEM((SUPK, ROW), jnp.int32),
            pltpu.VMEM((ROW,), jnp.float32),
            pltpu.VMEM_SHARED((NPAD,), jnp.float32),
        ],
    )(dst2d)


def _sc_scatter(src2d, dst2d, g):
    def body(src_hbm, dst_hbm, g_hbm, out_hbm,
             sidx0, didx0, gbuf0, isem0, sem0,
             sidx1, didx1, gbuf1, isem1, sem1, acc_sh):
        cid = lax.axis_index("c")
        tid = lax.axis_index("s")
        wid = cid * NS + tid
        base = wid * RW

        def zrow(i, c):
            gbuf0[i, :] = jnp.zeros((F,), jnp.float32)
            return c

        lax.fori_loop(0, SUPK * ROW, zrow, 0)
        nz = STRIPE // (SUPK * ROW)
        for t in range(nz):
            pltpu.sync_copy(
                gbuf0, acc_sh.at[pl.ds(tid * STRIPE + t * SUPK * ROW, SUPK * ROW)])
        rem = STRIPE - nz * SUPK * ROW
        if rem:
            pltpu.sync_copy(
                gbuf0.at[pl.ds(0, rem)],
                acc_sh.at[pl.ds(tid * STRIPE + nz * SUPK * ROW, rem)])
        plsc.subcore_barrier()

        def load_idx(s, sidx, didx, isem):
            r0 = base + s * SUPK
            pltpu.async_copy(src_hbm.at[pl.ds(r0, SUPK)], sidx, isem)
            pltpu.async_copy(dst_hbm.at[pl.ds(r0, SUPK)], didx, isem)

        def wait_idx(s, sidx, didx, isem):
            r0 = base + s * SUPK
            pltpu.make_async_copy(src_hbm.at[pl.ds(r0, SUPK)], sidx, isem).wait()
            pltpu.make_async_copy(dst_hbm.at[pl.ds(r0, SUPK)], didx, isem).wait()

        def fire(sidx, gbuf, sem):
            for j in range(SUPK):
                pltpu.async_copy(g_hbm.at[sidx.at[j]],
                                 gbuf.at[pl.ds(j * ROW, ROW)], sem)

        def drain(sidx, didx, gbuf, sem):
            for j in range(SUPK):
                pltpu.make_async_copy(g_hbm.at[sidx.at[j]],
                                      gbuf.at[pl.ds(j * ROW, ROW)], sem).wait()
            for j in range(SUPK):
                pltpu.sync_copy(gbuf.at[pl.ds(j * ROW, ROW)],
                                acc_sh.at[didx.at[j]], add=True)

        load_idx(0, sidx0, didx0, isem0)
        wait_idx(0, sidx0, didx0, isem0)
        fire(sidx0, gbuf0, sem0)
        load_idx(1, sidx1, didx1, isem1)

        def step(s, c):
            def run(sa, da, ga, ia, sma, sb, db, gb, ib, smb):
                wait_idx(s, sa, da, ia)
                fire(sa, ga, sma)
                drain(sb, db, gb, smb)

                @pl.when(s + 1 < NSUP)
                def _():
                    load_idx(s + 1, sb, db, ib)

            @pl.when(s % 2 == 1)
            def _():
                run(sidx1, didx1, gbuf1, isem1, sem1,
                    sidx0, didx0, gbuf0, isem0, sem0)

            @pl.when(s % 2 == 0)
            def _():
                run(sidx0, didx0, gbuf0, isem0, sem0,
                    sidx1, didx1, gbuf1, isem1, sem1)

            return c

        lax.fori_loop(1, NSUP, step, 0)
        if (NSUP - 1) % 2 == 0:
            drain(sidx0, didx0, gbuf0, sem0)
        else:
            drain(sidx1, didx1, gbuf1, sem1)
        plsc.subcore_barrier()
        pltpu.sync_copy(acc_sh.at[pl.ds(tid * STRIPE, STRIPE)],
                        out_hbm.at[cid, pl.ds(tid * STRIPE, STRIPE)])

    return pl.kernel(
        body,
        out_type=jax.ShapeDtypeStruct((NC, NPAD, F), jnp.float32),
        mesh=_sc_mesh(),
        compiler_params=pltpu.CompilerParams(use_tc_tiling_on_sc=False),
        scratch_types=[
            pltpu.VMEM((SUPK, ROW), jnp.int32),
            pltpu.VMEM((SUPK, ROW), jnp.int32),
            pltpu.VMEM((SUPK * ROW, F), jnp.float32),
            pltpu.SemaphoreType.DMA,
            pltpu.SemaphoreType.DMA,
            pltpu.VMEM((SUPK, ROW), jnp.int32),
            pltpu.VMEM((SUPK, ROW), jnp.int32),
            pltpu.VMEM((SUPK * ROW, F), jnp.float32),
            pltpu.SemaphoreType.DMA,
            pltpu.SemaphoreType.DMA,
            pltpu.VMEM_SHARED((NPAD, F), jnp.float32),
        ],
    )(src2d, dst2d, g)


def _dinv_block(degp):
    return lax.rsqrt(degp[0, :, 0] + degp[1, :, 0] + 1.0)


def _tc_embed(x, S, R, We1a, We1b, be1, We2, be2, Wg1, degp):
    def body(x_ref, s_ref, r_ref, we1a, we1b, be1_r, we2, be2_r, wg1, degp_ref,
             g_ref):
        q = r_ref[...] @ we1b[...]
        h = jnp.tanh(x_ref[...] @ we1a[...] + s_ref[...] @ q + be1_r[...])
        h = jnp.tanh(h @ we2[...] + be2_r[...])
        dinv = _dinv_block(degp_ref[...])
        g_ref[...] = (h @ wg1[...]) * dinv[:, None]

    return pl.pallas_call(
        body,
        grid=(GRID,),
        in_specs=[
            pl.BlockSpec((BLK, 42), lambda i: (i, 0)),
            pl.BlockSpec((BLK, 128), lambda i: (i, 0)),
            pl.BlockSpec((128, 3), lambda i: (0, 0)),
            pl.BlockSpec((42, 32), lambda i: (0, 0)),
            pl.BlockSpec((3, 32), lambda i: (0, 0)),
            pl.BlockSpec((1, 32), lambda i: (0, 0)),
            pl.BlockSpec((32, F), lambda i: (0, 0)),
            pl.BlockSpec((1, F), lambda i: (0, 0)),
            pl.BlockSpec((F, F), lambda i: (0, 0)),
            pl.BlockSpec((NC, BLK, 1), lambda i: (0, i, 0)),
        ],
        out_specs=pl.BlockSpec((BLK, F), lambda i: (i, 0)),
        out_shape=jax.ShapeDtypeStruct((N_NODES, F), jnp.float32),
    )(x, S, R, We1a, We1b, be1, We2, be2, Wg1, degp)


def _tc_mid(accp, g1, degp, bg1, Wg2):
    def body(accp_ref, g1_ref, degp_ref, bg1_r, wg2, g2_ref):
        dinv = _dinv_block(degp_ref[...])
        acc = accp_ref[0, :, :] + accp_ref[1, :, :] + g1_ref[...]
        h = jax.nn.relu(acc * dinv[:, None] + bg1_r[...])
        g2_ref[...] = (h @ wg2[...]) * dinv[:, None]

    return pl.pallas_call(
        body,
        grid=(GRID,),
        in_specs=[
            pl.BlockSpec((NC, BLK, F), lambda i: (0, i, 0)),
            pl.BlockSpec((BLK, F), lambda i: (i, 0)),
            pl.BlockSpec((NC, BLK, 1), lambda i: (0, i, 0)),
            pl.BlockSpec((1, F), lambda i: (0, 0)),
            pl.BlockSpec((F, F), lambda i: (0, 0)),
        ],
        out_specs=pl.BlockSpec((BLK, F), lambda i: (i, 0)),
        out_shape=jax.ShapeDtypeStruct((N_NODES, F), jnp.float32),
    )(accp, g1, degp, bg1, Wg2)


def _tc_pred(accp, g2, degp, bg2, Wp1, bp1, Wp2, bp2, priors):
    def body(accp_ref, g2_ref, degp_ref, bg2_r, wp1, bp1_r, wp2, bp2_r, pri_ref,
             out_ref):
        dinv = _dinv_block(degp_ref[...])
        acc = accp_ref[0, :, :] + accp_ref[1, :, :] + g2_ref[...]
        h = jax.nn.relu(acc * dinv[:, None] + bg2_r[...])
        o = jnp.tanh(h @ wp1[...] + bp1_r[...])
        o = jnp.tanh(o @ wp2[...] + bp2_r[...])
        out_ref[...] = o + pri_ref[...]

    return pl.pallas_call(
        body,
        grid=(GRID,),
        in_specs=[
            pl.BlockSpec((NC, BLK, F), lambda i: (0, i, 0)),
            pl.BlockSpec((BLK, F), lambda i: (i, 0)),
            pl.BlockSpec((NC, BLK, 1), lambda i: (0, i, 0)),
            pl.BlockSpec((1, F), lambda i: (0, 0)),
            pl.BlockSpec((F, 8), lambda i: (0, 0)),
            pl.BlockSpec((1, 8), lambda i: (0, 0)),
            pl.BlockSpec((8, 1), lambda i: (0, 0)),
            pl.BlockSpec((1, 1), lambda i: (0, 0)),
            pl.BlockSpec((BLK, 1), lambda i: (i, 0)),
        ],
        out_specs=pl.BlockSpec((BLK, 1), lambda i: (i, 0)),
        out_shape=jax.ShapeDtypeStruct((N_NODES, 1), jnp.float32),
    )(accp, g2, degp, bg2, Wp1, bp1, Wp2, bp2, priors)


def kernel(x, edge_index, priors, S, R, We1, be1, We2, be2, Wg1, bg1, Wg2, bg2,
           Wp1, bp1, Wp2, bp2):
    src = edge_index[0].astype(jnp.int32)
    dst = edge_index[1].astype(jnp.int32)
    pad = PE - N_EDGES
    src2d = jnp.concatenate([src, jnp.zeros((pad,), jnp.int32)]).reshape(-1, ROW)
    dst2d = jnp.concatenate([dst, jnp.full((pad,), DUMP, jnp.int32)]).reshape(-1, ROW)

    degp = _sc_degree(dst2d).reshape(NC, NPAD, 1)
    g1 = _tc_embed(x, S, R, We1[:42], We1[42:], be1.reshape(1, -1),
                   We2, be2.reshape(1, -1), Wg1, degp)
    acc1 = _sc_scatter(src2d, dst2d, g1)
    g2 = _tc_mid(acc1, g1, degp, bg1.reshape(1, -1), Wg2)
    acc2 = _sc_scatter(src2d, dst2d, g2)
    return _tc_pred(acc2, g2, degp, bg2.reshape(1, -1), Wp1, bp1.reshape(1, -1),
                    Wp2, bp2.reshape(1, -1), priors)


# speedup vs baseline: 56.9347x; 1.0084x over previous
import functools

import jax
import jax.numpy as jnp
from jax import lax
from jax.experimental import pallas as pl
from jax.experimental.pallas import tpu as pltpu
from jax.experimental.pallas import tpu_sc as plsc

N_NODES = 100000
N_EDGES = 3200000
F = 16
NC, NS = 2, 16
NW = NC * NS
ROW = 128
SUPK = 6
RW = 786
NSUP = RW // SUPK
PE = NW * RW * ROW
STRIPE = 6272
NPAD = NS * STRIPE
DUMP = N_NODES
BLK = 5000
GRID = N_NODES // BLK


def _sc_mesh():
    return plsc.VectorSubcoreMesh(core_axis_name="c", subcore_axis_name="s")


def _sc_degree(dst2d):
    def body(dst_hbm, out_hbm, didx, buf, deg_sh):
        cid = lax.axis_index("c")
        tid = lax.axis_index("s")
        wid = cid * NS + tid

        def fill(val):
            def w(i, c):
                buf[pl.ds(i * F, F)] = jnp.full((F,), val, jnp.float32)
                return c
            lax.fori_loop(0, ROW // F, w, 0)

        fill(0.0)

        def zcopy(t, c):
            pltpu.sync_copy(buf, deg_sh.at[pl.ds(tid * STRIPE + t * ROW, ROW)])
            return c

        lax.fori_loop(0, STRIPE // ROW, zcopy, 0)
        fill(1.0)
        plsc.subcore_barrier()

        base = wid * RW

        def sup(s, c):
            pltpu.sync_copy(dst_hbm.at[pl.ds(base + s * SUPK, SUPK)], didx)
            for j in range(SUPK):
                pltpu.sync_copy(buf, deg_sh.at[didx.at[j]], add=True)
            return c

        lax.fori_loop(0, NSUP, sup, 0)
        plsc.subcore_barrier()
        pltpu.sync_copy(deg_sh.at[pl.ds(tid * STRIPE, STRIPE)],
                        out_hbm.at[cid, pl.ds(tid * STRIPE, STRIPE)])

    return pl.kernel(
        body,
        out_type=jax.ShapeDtypeStruct((NC, NPAD), jnp.float32),
        mesh=_sc_mesh(),
        compiler_params=pltpu.CompilerParams(use_tc_tiling_on_sc=False),
        scratch_types=[
            pltpu.VMEM((SUPK, ROW), jnp.int32),
            pltpu.VMEM((ROW,), jnp.float32),
            pltpu.VMEM_SHARED((NPAD,), jnp.float32),
        ],
    )(dst2d)


def _sc_scatter(src2d, dst2d, g):
    def body(src_hbm, dst_hbm, g_hbm, out_hbm,
             sidx0, didx0, gbuf0, isem0, sem0,
             sidx1, didx1, gbuf1, isem1, sem1, acc_sh):
        cid = lax.axis_index("c")
        tid = lax.axis_index("s")
        wid = cid * NS + tid
        base = wid * RW

        def zrow(i, c):
            gbuf0[i, :] = jnp.zeros((F,), jnp.float32)
            return c

        lax.fori_loop(0, SUPK * ROW, zrow, 0)
        nz = STRIPE // (SUPK * ROW)
        for t in range(nz):
            pltpu.sync_copy(
                gbuf0, acc_sh.at[pl.ds(tid * STRIPE + t * SUPK * ROW, SUPK * ROW)])
        rem = STRIPE - nz * SUPK * ROW
        if rem:
            pltpu.sync_copy(
                gbuf0.at[pl.ds(0, rem)],
                acc_sh.at[pl.ds(tid * STRIPE + nz * SUPK * ROW, rem)])
        plsc.subcore_barrier()

        def load_idx(s, sidx, didx, isem):
            r0 = base + s * SUPK
            pltpu.async_copy(src_hbm.at[pl.ds(r0, SUPK)], sidx, isem)
            pltpu.async_copy(dst_hbm.at[pl.ds(r0, SUPK)], didx, isem)

        def wait_idx(s, sidx, didx, isem):
            r0 = base + s * SUPK
            pltpu.make_async_copy(src_hbm.at[pl.ds(r0, SUPK)], sidx, isem).wait()
            pltpu.make_async_copy(dst_hbm.at[pl.ds(r0, SUPK)], didx, isem).wait()

        def fire(sidx, gbuf, sem):
            for j in range(SUPK):
                pltpu.async_copy(g_hbm.at[sidx.at[j]],
                                 gbuf.at[pl.ds(j * ROW, ROW)], sem)

        def drain(sidx, didx, gbuf, sem):
            for j in range(SUPK):
                pltpu.make_async_copy(g_hbm.at[sidx.at[j]],
                                      gbuf.at[pl.ds(j * ROW, ROW)], sem).wait()
            for j in range(SUPK):
                pltpu.sync_copy(gbuf.at[pl.ds(j * ROW, ROW)],
                                acc_sh.at[didx.at[j]], add=True)

        load_idx(0, sidx0, didx0, isem0)
        wait_idx(0, sidx0, didx0, isem0)
        fire(sidx0, gbuf0, sem0)
        load_idx(1, sidx1, didx1, isem1)

        def step(s, c):
            def run(sa, da, ga, ia, sma, sb, db, gb, ib, smb):
                wait_idx(s, sa, da, ia)
                fire(sa, ga, sma)
                drain(sb, db, gb, smb)

                @pl.when(s + 1 < NSUP)
                def _():
                    load_idx(s + 1, sb, db, ib)

            @pl.when(s % 2 == 1)
            def _():
                run(sidx1, didx1, gbuf1, isem1, sem1,
                    sidx0, didx0, gbuf0, isem0, sem0)

            @pl.when(s % 2 == 0)
            def _():
                run(sidx0, didx0, gbuf0, isem0, sem0,
                    sidx1, didx1, gbuf1, isem1, sem1)

            return c

        lax.fori_loop(1, NSUP, step, 0)
        if (NSUP - 1) % 2 == 0:
            drain(sidx0, didx0, gbuf0, sem0)
        else:
            drain(sidx1, didx1, gbuf1, sem1)
        plsc.subcore_barrier()
        pltpu.sync_copy(acc_sh.at[pl.ds(tid * STRIPE, STRIPE)],
                        out_hbm.at[cid, pl.ds(tid * STRIPE, STRIPE)])

    return pl.kernel(
        body,
        out_type=jax.ShapeDtypeStruct((NC, NPAD, F), jnp.float32),
        mesh=_sc_mesh(),
        compiler_params=pltpu.CompilerParams(use_tc_tiling_on_sc=False),
        scratch_types=[
            pltpu.VMEM((SUPK, ROW), jnp.int32),
            pltpu.VMEM((SUPK, ROW), jnp.int32),
            pltpu.VMEM((SUPK * ROW, F), jnp.float32),
            pltpu.SemaphoreType.DMA,
            pltpu.SemaphoreType.DMA,
            pltpu.VMEM((SUPK, ROW), jnp.int32),
            pltpu.VMEM((SUPK, ROW), jnp.int32),
            pltpu.VMEM((SUPK * ROW, F), jnp.float32),
            pltpu.SemaphoreType.DMA,
            pltpu.SemaphoreType.DMA,
            pltpu.VMEM_SHARED((NPAD, F), jnp.float32),
        ],
    )(src2d, dst2d, g)


def _dinv_block(degp):
    return lax.rsqrt(degp[0, :, 0] + degp[1, :, 0] + 1.0)


def _tc_embed(x, S, R, We1a, We1b, be1, We2, be2, Wg1, degp):
    def body(x_ref, s_ref, r_ref, we1a, we1b, be1_r, we2, be2_r, wg1, degp_ref,
             g_ref):
        q = r_ref[...] @ we1b[...]
        h = jnp.tanh(x_ref[...] @ we1a[...] + s_ref[...] @ q + be1_r[...])
        h = jnp.tanh(h @ we2[...] + be2_r[...])
        dinv = _dinv_block(degp_ref[...])
        g_ref[...] = (h @ wg1[...]) * dinv[:, None]

    return pl.pallas_call(
        body,
        grid=(GRID,),
        in_specs=[
            pl.BlockSpec((BLK, 42), lambda i: (i, 0)),
            pl.BlockSpec((BLK, 128), lambda i: (i, 0)),
            pl.BlockSpec((128, 3), lambda i: (0, 0)),
            pl.BlockSpec((42, 32), lambda i: (0, 0)),
            pl.BlockSpec((3, 32), lambda i: (0, 0)),
            pl.BlockSpec((1, 32), lambda i: (0, 0)),
            pl.BlockSpec((32, F), lambda i: (0, 0)),
            pl.BlockSpec((1, F), lambda i: (0, 0)),
            pl.BlockSpec((F, F), lambda i: (0, 0)),
            pl.BlockSpec((NC, BLK, 1), lambda i: (0, i, 0)),
        ],
        out_specs=pl.BlockSpec((BLK, F), lambda i: (i, 0)),
        out_shape=jax.ShapeDtypeStruct((N_NODES, F), jnp.float32),
    )(x, S, R, We1a, We1b, be1, We2, be2, Wg1, degp)


def _tc_mid(accp, g1, degp, bg1, Wg2):
    def body(accp_ref, g1_ref, degp_ref, bg1_r, wg2, g2_ref):
        dinv = _dinv_block(degp_ref[...])
        acc = accp_ref[0, :, :] + accp_ref[1, :, :] + g1_ref[...]
        h = jax.nn.relu(acc * dinv[:, None] + bg1_r[...])
        g2_ref[...] = (h @ wg2[...]) * dinv[:, None]

    return pl.pallas_call(
        body,
        grid=(GRID,),
        in_specs=[
            pl.BlockSpec((NC, BLK, F), lambda i: (0, i, 0)),
            pl.BlockSpec((BLK, F), lambda i: (i, 0)),
            pl.BlockSpec((NC, BLK, 1), lambda i: (0, i, 0)),
            pl.BlockSpec((1, F), lambda i: (0, 0)),
            pl.BlockSpec((F, F), lambda i: (0, 0)),
        ],
        out_specs=pl.BlockSpec((BLK, F), lambda i: (i, 0)),
        out_shape=jax.ShapeDtypeStruct((N_NODES, F), jnp.float32),
    )(accp, g1, degp, bg1, Wg2)


def _tc_pred(accp, g2, degp, bg2, Wp1, bp1, Wp2, bp2, priors):
    def body(accp_ref, g2_ref, degp_ref, bg2_r, wp1, bp1_r, wp2, bp2_r, pri_ref,
             out_ref):
        dinv = _dinv_block(degp_ref[...])
        acc = accp_ref[0, :, :] + accp_ref[1, :, :] + g2_ref[...]
        h = jax.nn.relu(acc * dinv[:, None] + bg2_r[...])
        o = jnp.tanh(h @ wp1[...] + bp1_r[...])
        o = jnp.tanh(o @ wp2[...] + bp2_r[...])
        out_ref[...] = o + pri_ref[...]

    return pl.pallas_call(
        body,
        grid=(GRID,),
        in_specs=[
            pl.BlockSpec((NC, BLK, F), lambda i: (0, i, 0)),
            pl.BlockSpec((BLK, F), lambda i: (i, 0)),
            pl.BlockSpec((NC, BLK, 1), lambda i: (0, i, 0)),
            pl.BlockSpec((1, F), lambda i: (0, 0)),
            pl.BlockSpec((F, 8), lambda i: (0, 0)),
            pl.BlockSpec((1, 8), lambda i: (0, 0)),
            pl.BlockSpec((8, 1), lambda i: (0, 0)),
            pl.BlockSpec((1, 1), lambda i: (0, 0)),
            pl.BlockSpec((BLK, 1), lambda i: (i, 0)),
        ],
        out_specs=pl.BlockSpec((BLK, 1), lambda i: (i, 0)),
        out_shape=jax.ShapeDtypeStruct((N_NODES, 1), jnp.float32),
    )(accp, g2, degp, bg2, Wp1, bp1, Wp2, bp2, priors)


def kernel(x, edge_index, priors, S, R, We1, be1, We2, be2, Wg1, bg1, Wg2, bg2,
           Wp1, bp1, Wp2, bp2):
    src = edge_index[0].astype(jnp.int32)
    dst = edge_index[1].astype(jnp.int32)
    pad = PE - N_EDGES
    src2d = jnp.concatenate([src, jnp.zeros((pad,), jnp.int32)]).reshape(-1, ROW)
    dst2d = jnp.concatenate([dst, jnp.full((pad,), DUMP, jnp.int32)]).reshape(-1, ROW)

    degp = _sc_degree(dst2d).reshape(NC, NPAD, 1)
    g1 = _tc_embed(x, S, R, We1[:42], We1[42:], be1.reshape(1, -1),
                   We2, be2.reshape(1, -1), Wg1, degp)
    acc1 = _sc_scatter(src2d, dst2d, g1)
    g2 = _tc_mid(acc1, g1, degp, bg1.reshape(1, -1), Wg2)
    acc2 = _sc_scatter(src2d, dst2d, g2)
    return _tc_pred(acc2, g2, degp, bg2.reshape(1, -1), Wp1, bp1.reshape(1, -1),
                    Wp2, bp2.reshape(1, -1), priors)


# speedup vs baseline: 77.4219x; 1.3598x over previous
import functools

import jax
import jax.numpy as jnp
from jax import lax
from jax.experimental import pallas as pl
from jax.experimental.pallas import tpu as pltpu
from jax.experimental.pallas import tpu_sc as plsc

N_NODES = 100000
N_EDGES = 3200000
F = 16
PK = 8
NC, NS = 2, 16
NW = NC * NS
ROW = 128
SUPK = 6
RW = 786
NSUP = RW // SUPK
PE = NW * RW * ROW
STRIPE = 6272
NPAD = NS * STRIPE
DUMP = N_NODES
BLK = 6272
BLKP = BLK // PK
NP_N = N_NODES // PK
NP_A = NPAD // PK
GRID = NPAD // BLK


def _sc_mesh():
    return plsc.VectorSubcoreMesh(core_axis_name="c", subcore_axis_name="s")


def _sc_degree(dst2d):
    def body(dst_hbm, out_hbm, didx, buf, dbuf, ebuf, deg_sh):
        cid = lax.axis_index("c")
        tid = lax.axis_index("s")
        wid = cid * NS + tid

        def fill(val):
            def w(i, c):
                buf[pl.ds(i * F, F)] = jnp.full((F,), val, jnp.float32)
                return c
            lax.fori_loop(0, ROW // F, w, 0)

        fill(0.0)

        def zcopy(t, c):
            pltpu.sync_copy(buf, deg_sh.at[pl.ds(tid * STRIPE + t * ROW, ROW)])
            return c

        lax.fori_loop(0, STRIPE // ROW, zcopy, 0)
        fill(1.0)
        plsc.subcore_barrier()

        base = wid * RW

        def sup(s, c):
            pltpu.sync_copy(dst_hbm.at[pl.ds(base + s * SUPK, SUPK)], didx)
            for j in range(SUPK):
                pltpu.sync_copy(buf, deg_sh.at[didx.at[j]], add=True)
            return c

        lax.fori_loop(0, NSUP, sup, 0)
        plsc.subcore_barrier()

        def exp(t, c):
            r0 = tid * STRIPE + t * ROW
            pltpu.sync_copy(deg_sh.at[pl.ds(r0, ROW)], dbuf)

            def bk(k16, c2):
                v = dbuf[pl.ds(k16 * F, F)]
                for j in range(F):
                    ebuf[k16 * F + j, :] = jnp.full((F,), 1.0, jnp.float32) * v[j]
                return c2

            lax.fori_loop(0, ROW // F, bk, 0)
            pltpu.sync_copy(ebuf, out_hbm.at[cid, pl.ds(r0, ROW)])
            return c

        lax.fori_loop(0, STRIPE // ROW, exp, 0)

    return pl.kernel(
        body,
        out_type=jax.ShapeDtypeStruct((NC, NPAD, F), jnp.float32),
        mesh=_sc_mesh(),
        compiler_params=pltpu.CompilerParams(use_tc_tiling_on_sc=False),
        scratch_types=[
            pltpu.VMEM((SUPK, ROW), jnp.int32),
            pltpu.VMEM((ROW,), jnp.float32),
            pltpu.VMEM((ROW,), jnp.float32),
            pltpu.VMEM((ROW, F), jnp.float32),
            pltpu.VMEM_SHARED((NPAD,), jnp.float32),
        ],
    )(dst2d)


def _sc_scatter(src2d, dst2d, g):
    def body(src_hbm, dst_hbm, g_hbm, out_hbm,
             sidx0, didx0, gbuf0, isem0, sem0,
             sidx1, didx1, gbuf1, isem1, sem1, acc_sh):
        cid = lax.axis_index("c")
        tid = lax.axis_index("s")
        wid = cid * NS + tid
        base = wid * RW

        def zrow(i, c):
            gbuf0[i, :] = jnp.zeros((F,), jnp.float32)
            return c

        lax.fori_loop(0, SUPK * ROW, zrow, 0)
        nz = STRIPE // (SUPK * ROW)
        for t in range(nz):
            pltpu.sync_copy(
                gbuf0, acc_sh.at[pl.ds(tid * STRIPE + t * SUPK * ROW, SUPK * ROW)])
        rem = STRIPE - nz * SUPK * ROW
        if rem:
            pltpu.sync_copy(
                gbuf0.at[pl.ds(0, rem)],
                acc_sh.at[pl.ds(tid * STRIPE + nz * SUPK * ROW, rem)])
        plsc.subcore_barrier()

        def load_idx(s, sidx, didx, isem):
            r0 = base + s * SUPK
            pltpu.async_copy(src_hbm.at[pl.ds(r0, SUPK)], sidx, isem)
            pltpu.async_copy(dst_hbm.at[pl.ds(r0, SUPK)], didx, isem)

        def wait_idx(s, sidx, didx, isem):
            r0 = base + s * SUPK
            pltpu.make_async_copy(src_hbm.at[pl.ds(r0, SUPK)], sidx, isem).wait()
            pltpu.make_async_copy(dst_hbm.at[pl.ds(r0, SUPK)], didx, isem).wait()

        def fire(sidx, gbuf, sem):
            for j in range(SUPK):
                pltpu.async_copy(g_hbm.at[sidx.at[j]],
                                 gbuf.at[pl.ds(j * ROW, ROW)], sem)

        def drain(sidx, didx, gbuf, sem):
            for j in range(SUPK):
                pltpu.make_async_copy(g_hbm.at[sidx.at[j]],
                                      gbuf.at[pl.ds(j * ROW, ROW)], sem).wait()
            for j in range(SUPK):
                pltpu.sync_copy(gbuf.at[pl.ds(j * ROW, ROW)],
                                acc_sh.at[didx.at[j]], add=True)

        load_idx(0, sidx0, didx0, isem0)
        wait_idx(0, sidx0, didx0, isem0)
        fire(sidx0, gbuf0, sem0)
        load_idx(1, sidx1, didx1, isem1)

        def step(s, c):
            def run(sa, da, ga, ia, sma, sb, db, gb, ib, smb):
                wait_idx(s, sa, da, ia)
                fire(sa, ga, sma)
                drain(sb, db, gb, smb)

                @pl.when(s + 1 < NSUP)
                def _():
                    load_idx(s + 1, sb, db, ib)

            @pl.when(s % 2 == 1)
            def _():
                run(sidx1, didx1, gbuf1, isem1, sem1,
                    sidx0, didx0, gbuf0, isem0, sem0)

            @pl.when(s % 2 == 0)
            def _():
                run(sidx0, didx0, gbuf0, isem0, sem0,
                    sidx1, didx1, gbuf1, isem1, sem1)

            return c

        lax.fori_loop(1, NSUP, step, 0)
        if (NSUP - 1) % 2 == 0:
            drain(sidx0, didx0, gbuf0, sem0)
        else:
            drain(sidx1, didx1, gbuf1, sem1)
        plsc.subcore_barrier()
        pltpu.sync_copy(acc_sh.at[pl.ds(tid * STRIPE, STRIPE)],
                        out_hbm.at[cid, pl.ds(tid * STRIPE, STRIPE)])

    return pl.kernel(
        body,
        out_type=jax.ShapeDtypeStruct((NC, NPAD, F), jnp.float32),
        mesh=_sc_mesh(),
        compiler_params=pltpu.CompilerParams(use_tc_tiling_on_sc=False),
        scratch_types=[
            pltpu.VMEM((SUPK, ROW), jnp.int32),
            pltpu.VMEM((SUPK, ROW), jnp.int32),
            pltpu.VMEM((SUPK * ROW, F), jnp.float32),
            pltpu.SemaphoreType.DMA,
            pltpu.SemaphoreType.DMA,
            pltpu.VMEM((SUPK, ROW), jnp.int32),
            pltpu.VMEM((SUPK, ROW), jnp.int32),
            pltpu.VMEM((SUPK * ROW, F), jnp.float32),
            pltpu.SemaphoreType.DMA,
            pltpu.SemaphoreType.DMA,
            pltpu.VMEM_SHARED((NPAD, F), jnp.float32),
        ],
    )(src2d, dst2d, g)


def _dinv_packed(degp):
    return lax.rsqrt(degp[0] + degp[1] + 1.0)


def _tc_embed(x, S, R, We1a, We1b, be1, We2, be2, Wg1):
    def body(x_ref, s_ref, r_ref, we1a, we1b, be1_r, we2, be2_r, wg1, g_ref):
        q = r_ref[...] @ we1b[...]
        h = jnp.tanh(x_ref[...] @ we1a[...] + s_ref[...] @ q + be1_r[...])
        h = jnp.tanh(h @ we2[...] + be2_r[...])
        g_ref[...] = h @ wg1[...]

    return pl.pallas_call(
        body,
        grid=(GRID,),
        in_specs=[
            pl.BlockSpec((BLK, 42), lambda i: (i, 0)),
            pl.BlockSpec((BLK, 128), lambda i: (i, 0)),
            pl.BlockSpec((128, 3), lambda i: (0, 0)),
            pl.BlockSpec((42, 32), lambda i: (0, 0)),
            pl.BlockSpec((3, 32), lambda i: (0, 0)),
            pl.BlockSpec((1, 32), lambda i: (0, 0)),
            pl.BlockSpec((32, F), lambda i: (0, 0)),
            pl.BlockSpec((1, F), lambda i: (0, 0)),
            pl.BlockSpec((F, F), lambda i: (0, 0)),
        ],
        out_specs=pl.BlockSpec((BLK, F), lambda i: (i, 0)),
        out_shape=jax.ShapeDtypeStruct((NPAD, F), jnp.float32),
    )(x, S, R, We1a, We1b, be1, We2, be2, Wg1)


def _tc_mid(accp, g1, degp, bg1t, Wg2b):
    def body(accp_ref, g1_ref, degp_ref, bg1_r, wg2, g2_ref):
        dinv = _dinv_packed(degp_ref[...])
        acc = accp_ref[0] + accp_ref[1] + g1_ref[...]
        h = jax.nn.relu(acc * dinv + bg1_r[...])
        g2_ref[...] = (h @ wg2[...]) * dinv

    return pl.pallas_call(
        body,
        grid=(GRID,),
        in_specs=[
            pl.BlockSpec((NC, BLKP, PK * F), lambda i: (0, i, 0)),
            pl.BlockSpec((BLKP, PK * F), lambda i: (i, 0)),
            pl.BlockSpec((NC, BLKP, PK * F), lambda i: (0, i, 0)),
            pl.BlockSpec((1, PK * F), lambda i: (0, 0)),
            pl.BlockSpec((PK * F, PK * F), lambda i: (0, 0)),
        ],
        out_specs=pl.BlockSpec((BLKP, PK * F), lambda i: (i, 0)),
        out_shape=jax.ShapeDtypeStruct((NP_A, PK * F), jnp.float32),
    )(accp, g1, degp, bg1t, Wg2b)


def _tc_pred(accp, g2, degp, bg2t, Wp1b, bp1t, Wp2b, bp2t, priorsp):
    def body(accp_ref, g2_ref, degp_ref, bg2_r, wp1, bp1_r, wp2, bp2_r, pri_ref,
             out_ref):
        dinv = _dinv_packed(degp_ref[...])
        acc = accp_ref[0] + accp_ref[1] + g2_ref[...]
        h = jax.nn.relu(acc * dinv + bg2_r[...])
        o = jnp.tanh(h @ wp1[...] + bp1_r[...])
        o = jnp.tanh(o @ wp2[...] + bp2_r[...])
        out_ref[...] = o + pri_ref[...]

    return pl.pallas_call(
        body,
        grid=(GRID,),
        in_specs=[
            pl.BlockSpec((NC, BLKP, PK * F), lambda i: (0, i, 0)),
            pl.BlockSpec((BLKP, PK * F), lambda i: (i, 0)),
            pl.BlockSpec((NC, BLKP, PK * F), lambda i: (0, i, 0)),
            pl.BlockSpec((1, PK * F), lambda i: (0, 0)),
            pl.BlockSpec((PK * F, PK * 8), lambda i: (0, 0)),
            pl.BlockSpec((1, PK * 8), lambda i: (0, 0)),
            pl.BlockSpec((PK * 8, PK), lambda i: (0, 0)),
            pl.BlockSpec((1, PK), lambda i: (0, 0)),
            pl.BlockSpec((BLKP, PK), lambda i: (i, 0)),
        ],
        out_specs=pl.BlockSpec((BLKP, PK), lambda i: (i, 0)),
        out_shape=jax.ShapeDtypeStruct((NP_N, PK), jnp.float32),
    )(accp, g2, degp, bg2t, Wp1b, bp1t, Wp2b, bp2t, priorsp)


def kernel(x, edge_index, priors, S, R, We1, be1, We2, be2, Wg1, bg1, Wg2, bg2,
           Wp1, bp1, Wp2, bp2):
    src = edge_index[0].astype(jnp.int32)
    dst = edge_index[1].astype(jnp.int32)
    pad = PE - N_EDGES
    src2d = jnp.concatenate([src, jnp.zeros((pad,), jnp.int32)]).reshape(-1, ROW)
    dst2d = jnp.concatenate([dst, jnp.full((pad,), DUMP, jnp.int32)]).reshape(-1, ROW)

    eye8 = jnp.eye(PK, dtype=jnp.float32)
    Wg2b = jnp.kron(eye8, Wg2)
    Wp1b = jnp.kron(eye8, Wp1)
    Wp2b = jnp.kron(eye8, Wp2)
    bg1t = jnp.tile(bg1, PK).reshape(1, -1)
    bg2t = jnp.tile(bg2, PK).reshape(1, -1)
    bp1t = jnp.tile(bp1, PK).reshape(1, -1)
    bp2t = jnp.tile(bp2, PK).reshape(1, -1)
    priorsp = priors.reshape(NP_N, PK)

    degp = _sc_degree(dst2d)
    degpk = degp.reshape(NC, NP_A, PK * F)
    hw1 = _tc_embed(x, S, R, We1[:42], We1[42:], be1.reshape(1, -1),
                    We2, be2.reshape(1, -1), Wg1)
    dinv_lin = lax.rsqrt(degp[0] + degp[1] + 1.0)
    g1 = hw1 * dinv_lin
    acc1 = _sc_scatter(src2d, dst2d, g1)
    g1p = g1.reshape(NP_A, PK * F)
    g2 = _tc_mid(acc1.reshape(NC, NP_A, PK * F), g1p, degpk, bg1t, Wg2b)
    acc2 = _sc_scatter(src2d, dst2d, g2.reshape(NPAD, F))
    outp = _tc_pred(acc2.reshape(NC, NP_A, PK * F), g2, degpk, bg2t,
                    Wp1b, bp1t, Wp2b, bp2t, priorsp)
    return outp.reshape(N_NODES, 1)
